# baseline (speedup 1.0000x reference)
import math

import jax
import jax.numpy as jnp
import numpy as np
from jax import lax
from jax.experimental import pallas as pl
from jax.experimental.pallas import tpu as pltpu
from jax.experimental.pallas import tpu_sc as plsc

VOCAB = 1000000
MAX_LEN = 512
DIM = 64
BATCH = 1024
SEQ = 200

NW = 32
ROWS = BATCH * SEQ
RPW = ROWS // NW
CHUNK = 128
NCH = RPW // CHUNK


def _pos_table():
    den = np.exp(-np.arange(0, DIM, 2, dtype=np.float64) * math.log(10000.0) / DIM)
    pos = np.arange(0, SEQ, dtype=np.float64).reshape(SEQ, 1)
    pe = np.zeros((SEQ, DIM), dtype=np.float64)
    pe[:, 0::2] = np.sin(pos * den)
    pe[:, 1::2] = np.cos(pos * den)
    return jnp.asarray(np.concatenate([pe, pe], axis=0), dtype=jnp.float32)


def _body(table_hbm, tok_hbm, pos_hbm, out_hbm, idx_v, pos_v, buf, sem_g):
    wid = lax.axis_index("s") * 2 + lax.axis_index("c")

    pltpu.sync_copy(tok_hbm.at[wid], idx_v)
    pltpu.sync_copy(pos_hbm, pos_v)

    def chunk_body(c, carry):
        base = wid * RPW + c * CHUNK
        pltpu.async_copy(table_hbm.at[idx_v.at[c]], buf, sem_g).wait()
        pbase = lax.rem(c * CHUNK, SEQ)

        def row_body(r, carry2):
            pr = pbase + r
            for t in range(4):
                sl = pl.ds(t * 16, 16)
                buf[r, sl] = buf[r, sl] + pos_v[pr, sl]
            return carry2

        lax.fori_loop(0, CHUNK, row_body, 0)
        pltpu.sync_copy(buf, out_hbm.at[pl.ds(base, CHUNK)])
        return carry

    lax.fori_loop(0, NCH, chunk_body, 0)


def kernel(tokens, table):
    tok = tokens.astype(jnp.int32).reshape(NW, NCH, CHUNK)
    pos = _pos_table()

    mesh = plsc.VectorSubcoreMesh(core_axis_name="c", subcore_axis_name="s")
    run = pl.kernel(
        _body,
        mesh=mesh,
        compiler_params=pltpu.CompilerParams(use_tc_tiling_on_sc=False),
        out_type=jax.ShapeDtypeStruct((ROWS, DIM), jnp.float32),
        scratch_types=[
            pltpu.VMEM((NCH, CHUNK), jnp.int32),
            pltpu.VMEM((2 * SEQ, DIM), jnp.float32),
            pltpu.VMEM((CHUNK, DIM), jnp.float32),
            pltpu.SemaphoreType.DMA,
        ],
    )
    out = run(table, tok, pos)
    return out.reshape(BATCH, SEQ, DIM)


# speedup vs baseline: 1.1907x; 1.1907x over previous
import math

import jax
import jax.numpy as jnp
import numpy as np
from jax import lax
from jax.experimental import pallas as pl
from jax.experimental.pallas import tpu as pltpu
from jax.experimental.pallas import tpu_sc as plsc

VOCAB = 1000000
MAX_LEN = 512
DIM = 64
BATCH = 1024
SEQ = 200

NW = 32
ROWS = BATCH * SEQ
RPW = ROWS // NW
CHUNK = 128
NCH = RPW // CHUNK
NB = 5
NG = NCH // NB


def _pos_table():
    den = np.exp(-np.arange(0, DIM, 2, dtype=np.float64) * math.log(10000.0) / DIM)
    pos = np.arange(0, SEQ, dtype=np.float64).reshape(SEQ, 1)
    pe = np.zeros((SEQ, DIM), dtype=np.float64)
    pe[:, 0::2] = np.sin(pos * den)
    pe[:, 1::2] = np.cos(pos * den)
    return jnp.asarray(np.concatenate([pe, pe], axis=0), dtype=jnp.float32)


def _body(table_hbm, tok_hbm, pos_hbm, out_hbm, idx_v, pos_v, *bufs_and_sems):
    bufs = bufs_and_sems[:NB]
    sem_g = bufs_and_sems[NB:2 * NB]
    sem_o = bufs_and_sems[2 * NB:3 * NB]

    wid = lax.axis_index("s") * 2 + lax.axis_index("c")
    obase = wid * RPW

    pltpu.sync_copy(tok_hbm.at[wid], idx_v)
    pltpu.sync_copy(pos_hbm, pos_v)

    def start_gather(c, b):
        pltpu.async_copy(table_hbm.at[idx_v.at[c]], bufs[b], sem_g[b])

    def wait_gather(c, b):
        pltpu.make_async_copy(table_hbm.at[idx_v.at[c]], bufs[b], sem_g[b]).wait()

    def start_out(c, b):
        pltpu.async_copy(bufs[b], out_hbm.at[pl.ds(obase + c * CHUNK, CHUNK)], sem_o[b])

    def wait_out(c, b):
        pltpu.make_async_copy(
            bufs[b], out_hbm.at[pl.ds(obase + c * CHUNK, CHUNK)], sem_o[b]
        ).wait()

    def add_pos(c, b):
        buf = bufs[b]
        pbase = lax.rem(c * CHUNK, SEQ)

        @plsc.parallel_loop(0, CHUNK, step=1, unroll=8)
        def _add(r):
            pr = pbase + r
            for t in range(4):
                sl = pl.ds(t * 16, 16)
                buf[r, sl] = buf[r, sl] + pos_v[pr, sl]

    for b in range(NB - 1):
        start_gather(b, b)

    def step(c, b, first, issue_ahead=True):
        bn = (b - 1) % NB
        if issue_ahead:
            if not first:
                wait_out(c - 1, bn)
            start_gather(c + NB - 1, bn)
        wait_gather(c, b)
        add_pos(c, b)
        start_out(c, b)

    for b in range(NB):
        step(b, b, first=(b == 0))

    def group(g, carry):
        for b in range(NB):
            step(g * NB + b, b, first=False)
        return carry

    lax.fori_loop(1, NG - 1, group, 0)
    for b in range(NB):
        c = (NG - 1) * NB + b
        step(c, b, first=False, issue_ahead=(c + NB - 1 < NCH))

    for b in range(NB):
        wait_out((NG - 1) * NB + b, b)


def kernel(tokens, table):
    tok = tokens.astype(jnp.int32).reshape(NW, NCH, CHUNK)
    pos = _pos_table()

    mesh = plsc.VectorSubcoreMesh(core_axis_name="c", subcore_axis_name="s")
    run = pl.kernel(
        _body,
        mesh=mesh,
        compiler_params=pltpu.CompilerParams(use_tc_tiling_on_sc=False),
        out_type=jax.ShapeDtypeStruct((ROWS, DIM), jnp.float32),
        scratch_types=(
            [pltpu.VMEM((NCH, CHUNK), jnp.int32),
             pltpu.VMEM((2 * SEQ, DIM), jnp.float32)]
            + [pltpu.VMEM((CHUNK, DIM), jnp.float32) for _ in range(NB)]
            + [pltpu.SemaphoreType.DMA for _ in range(2 * NB)]
        ),
    )
    out = run(table, tok, pos)
    return out.reshape(BATCH, SEQ, DIM)


# speedup vs baseline: 1.1968x; 1.0051x over previous
import math

import jax
import jax.numpy as jnp
import numpy as np
from jax import lax
from jax.experimental import pallas as pl
from jax.experimental.pallas import tpu as pltpu
from jax.experimental.pallas import tpu_sc as plsc

VOCAB = 1000000
MAX_LEN = 512
DIM = 64
BATCH = 1024
SEQ = 200

NW = 32
SPW = BATCH // NW
HALF = SEQ // 2
NB = 4
NG = SPW // NB


def _pos_table():
    den = np.exp(-np.arange(0, DIM, 2, dtype=np.float64) * math.log(10000.0) / DIM)
    pos = np.arange(0, SEQ, dtype=np.float64).reshape(SEQ, 1)
    pe = np.zeros((SEQ, DIM), dtype=np.float64)
    pe[:, 0::2] = np.sin(pos * den)
    pe[:, 1::2] = np.cos(pos * den)
    return jnp.asarray(pe, dtype=jnp.float32)


def _body(table_hbm, tok_hbm, pos_hbm, out_hbm, idx_v, pos_v, *bufs_and_sems):
    bufs = bufs_and_sems[:NB]
    sem_g = bufs_and_sems[NB:2 * NB]
    sem_o = bufs_and_sems[2 * NB:3 * NB]

    wid = lax.axis_index("s") * 2 + lax.axis_index("c")
    sbase = wid * SPW

    pltpu.sync_copy(tok_hbm.at[pl.ds(sbase, SPW)], idx_v)
    pltpu.sync_copy(pos_hbm, pos_v)

    def gather_halves(s, b):
        for h in range(2):
            yield pltpu.make_async_copy(
                table_hbm.at[idx_v.at[s, h]],
                bufs[b].at[pl.ds(h * HALF, HALF)],
                sem_g[b],
            )

    def start_gather(s, b):
        for cp in gather_halves(s, b):
            cp.start()

    def wait_gather(s, b):
        for cp in gather_halves(s, b):
            cp.wait()

    def start_out(s, b):
        pltpu.async_copy(bufs[b], out_hbm.at[sbase + s], sem_o[b])

    def wait_out(s, b):
        pltpu.make_async_copy(bufs[b], out_hbm.at[sbase + s], sem_o[b]).wait()

    def add_pos(b):
        buf = bufs[b]

        @plsc.parallel_loop(0, SEQ, step=1, unroll=8)
        def _add(r):
            for t in range(4):
                sl = pl.ds(t * 16, 16)
                buf[r, sl] = buf[r, sl] + pos_v[r, sl]

    for b in range(NB - 1):
        start_gather(b, b)

    def step(s, b, first, issue_ahead=True):
        bn = (b - 1) % NB
        if issue_ahead:
            if not first:
                wait_out(s - 1, bn)
            start_gather(s + NB - 1, bn)
        wait_gather(s, b)
        add_pos(b)
        start_out(s, b)

    for b in range(NB):
        step(b, b, first=(b == 0))

    def group(g, carry):
        for b in range(NB):
            step(g * NB + b, b, first=False)
        return carry

    lax.fori_loop(1, NG - 1, group, 0)
    for b in range(NB):
        s = (NG - 1) * NB + b
        step(s, b, first=False, issue_ahead=(s + NB - 1 < SPW))

    for b in range(NB):
        wait_out((NG - 1) * NB + b, b)


def kernel(tokens, table):
    tok = tokens.astype(jnp.int32).reshape(BATCH, 2, HALF)
    pos = _pos_table()

    mesh = plsc.VectorSubcoreMesh(core_axis_name="c", subcore_axis_name="s")
    run = pl.kernel(
        _body,
        mesh=mesh,
        compiler_params=pltpu.CompilerParams(use_tc_tiling_on_sc=False),
        out_type=jax.ShapeDtypeStruct((BATCH, SEQ, DIM), jnp.float32),
        scratch_types=(
            [pltpu.VMEM((SPW, 2, HALF), jnp.int32),
             pltpu.VMEM((SEQ, DIM), jnp.float32)]
            + [pltpu.VMEM((SEQ, DIM), jnp.float32) for _ in range(NB)]
            + [pltpu.SemaphoreType.DMA for _ in range(2 * NB)]
        ),
    )
    return run(table, tok, pos)
